# Initial kernel scaffold; baseline (speedup 1.0000x reference)
import jax
import jax.numpy as jnp
from jax.experimental import pallas as pl


def kernel(x, edge_index, edge_attr, batch, Wpre, bpre, Wedge, bedge, Wpost, bpost, Wlin, blin, bn_gamma, bn_beta, W1, b1, W2, b2, W3, b3):
    raise NotImplementedError("write your pallas kernel here")


# speedup vs baseline: 110.7706x; 110.7706x over previous
import functools
import math

import jax
import jax.numpy as jnp
import numpy as np
from jax import lax
from jax.experimental import pallas as pl
from jax.experimental.pallas import tpu as pltpu, tpu_sc as plsc

N = 50000
E = 800000
NG = 512
F = 5
T = 5
TF = T * F

_DEG = np.array([0, 0, 0, 0, 0, 0, 200, 400, 800, 1200, 1800, 2400, 3000,
                 3600, 4000, 4300, 4400, 4400, 4300, 4000, 3600, 3000, 2400,
                 1800, 1200, 800, 400, 200], dtype=np.float64)
AVG_LOG = float((np.log(np.arange(_DEG.shape[0]) + 1.0) * _DEG).sum() / _DEG.sum())

NB = 64
NPB = 784
NPAD = NB * NPB
EPT = 25600
EPAD = 32 * EPT
CAP = 820352
CH = 128
BLK = 2000
NBLK = 25
BIG = 3.0e38

_mesh = plsc.VectorSubcoreMesh(core_axis_name="c", subcore_axis_name="s")
_sc_params = pltpu.CompilerParams(
    needs_layout_passes=False, use_tc_tiling_on_sc=False)


def _wid():
    return lax.axis_index("s") * 2 + lax.axis_index("c")


def _bucket_of(d):
    return ((d >> 4) * 2675) >> 17


def _bucket_offsets(histv, offs, sizes, wid, with_pre):
    carry = jnp.int32(0)
    zero16 = jnp.zeros((16,), jnp.int32)
    for g in range(4):
        tot = zero16
        pre = zero16
        for t in range(32):
            hrow = histv[t, g * 16:(g + 1) * 16]
            tot = tot + hrow
            if with_pre:
                pre = pre + jnp.where(t < wid, hrow, zero16)
        padded = (tot + 15) & (-16)
        incl = plsc.cumsum(padded)
        excl = incl - padded + carry
        if with_pre:
            offs[pl.ds(g * 16, 16)] = excl + pre
        else:
            offs[pl.ds(g * 16, 16)] = excl
        if sizes is not None:
            sizes[pl.ds(g * 16, 16)] = tot
        carry = carry + incl[15]


@functools.partial(
    pl.kernel,
    mesh=_mesh,
    compiler_params=_sc_params,
    out_type=jax.ShapeDtypeStruct((32, 64), jnp.int32),
    scratch_types=[
        pltpu.VMEM((512,), jnp.int32),
        pltpu.VMEM((64,), jnp.int32),
    ],
)
def _k_hist(dst_hbm, out_hbm, dstv, hist):
    wid = _wid()

    def z(i, c):
        hist[pl.ds(i * 16, 16)] = jnp.zeros((16,), jnp.int32)
        return c

    lax.fori_loop(0, 4, z, 0)
    iota = lax.iota(jnp.int32, 16)

    def chunk(ci, c):
        pltpu.sync_copy(dst_hbm.at[pl.ds(wid * EPT + ci * 512, 512)], dstv)

        def grp(g, cc):
            d = dstv[pl.ds(g * 16, 16)]
            b = _bucket_of(d)
            cnt = jnp.zeros((16,), jnp.int32)
            rank = jnp.zeros((16,), jnp.int32)
            for j in range(16):
                eq = b == b[j]
                cnt = cnt + jnp.where(eq, 1, 0)
                rank = rank + jnp.where(eq & (iota > j), 1, 0)
            uniq = rank == 0
            old = plsc.load_gather(hist, [b], mask=uniq)
            plsc.store_scatter(hist, [b], old + cnt, mask=uniq)
            return cc

        lax.fori_loop(0, 32, grp, 0)
        return c

    lax.fori_loop(0, EPT // 512, chunk, 0)
    pltpu.sync_copy(hist, out_hbm.at[wid])


@functools.partial(
    pl.kernel,
    mesh=_mesh,
    compiler_params=_sc_params,
    out_type=[
        jax.ShapeDtypeStruct((CAP,), jnp.int32),
        jax.ShapeDtypeStruct((CAP,), jnp.int32),
        jax.ShapeDtypeStruct((CAP,), jnp.float32),
    ],
    scratch_types=[
        pltpu.VMEM((32, 64), jnp.int32),
        pltpu.VMEM((64,), jnp.int32),
        pltpu.VMEM((CH,), jnp.int32),
        pltpu.VMEM((CH,), jnp.int32),
        pltpu.VMEM((CH,), jnp.float32),
        pltpu.VMEM((CH,), jnp.int32),
        pltpu.VMEM((CH,), jnp.int32),
        pltpu.SemaphoreType.DMA,
    ],
)
def _k_scatter(src_hbm, dst_hbm, a_hbm, hist_hbm, srcB, dstB, aB,
               histv, offs, srcv, dstv, av, posv, dlv, sem):
    wid = _wid()
    pltpu.sync_copy(hist_hbm, histv)
    _bucket_offsets(histv, offs, None, wid, True)
    iota = lax.iota(jnp.int32, 16)

    def chunk(ci, c):
        base = wid * EPT + ci * CH
        pltpu.sync_copy(src_hbm.at[pl.ds(base, CH)], srcv)
        pltpu.sync_copy(dst_hbm.at[pl.ds(base, CH)], dstv)
        pltpu.sync_copy(a_hbm.at[pl.ds(base, CH)], av)
        for g in range(CH // 16):
            d = dstv[pl.ds(g * 16, 16)]
            b = _bucket_of(d)
            cnt = jnp.zeros((16,), jnp.int32)
            rank = jnp.zeros((16,), jnp.int32)
            for j in range(16):
                eq = b == b[j]
                cnt = cnt + jnp.where(eq, 1, 0)
                rank = rank + jnp.where(eq & (iota > j), 1, 0)
            uniq = rank == 0
            basev = plsc.load_gather(offs, [b])
            plsc.store_scatter(offs, [b], basev + cnt, mask=uniq)
            posv[pl.ds(g * 16, 16)] = basev + rank
            dlv[pl.ds(g * 16, 16)] = d - b * NPB
        c1 = pltpu.async_copy(srcv, srcB.at[posv], sem)
        c2 = pltpu.async_copy(dlv, dstB.at[posv], sem)
        c3 = pltpu.async_copy(av, aB.at[posv], sem)
        c1.wait()
        c2.wait()
        c3.wait()
        return c

    lax.fori_loop(0, EPT // CH, chunk, 0)


_ACC = (NPB + 1) * 32


@functools.partial(
    pl.kernel,
    mesh=_mesh,
    compiler_params=_sc_params,
    out_type=[jax.ShapeDtypeStruct((NPAD * 32,), jnp.float32)
              for _ in range(4)],
    scratch_types=[
        pltpu.VMEM((32, 64), jnp.int32),
        pltpu.VMEM((64,), jnp.int32),
        pltpu.VMEM((64,), jnp.int32),
        pltpu.VMEM((CH,), jnp.int32),
        pltpu.VMEM((CH,), jnp.int32),
        pltpu.VMEM((CH,), jnp.float32),
        pltpu.VMEM((CH, 32), jnp.float32),
        pltpu.VMEM((32,), jnp.float32),
        pltpu.VMEM((_ACC,), jnp.float32),
        pltpu.VMEM((_ACC,), jnp.float32),
        pltpu.VMEM((_ACC,), jnp.float32),
        pltpu.VMEM((_ACC,), jnp.float32),
        pltpu.SemaphoreType.DMA,
    ],
)
def _k_edge(srcB, dstB, aB, hist_hbm, u_hbm, q_hbm,
            sum_o, sq_o, mn_o, mx_o,
            histv, starts, sizes, srcv, dstv, av, rows, uvv,
            accS, accQ, accN, accX, sem):
    wid = _wid()
    pltpu.sync_copy(hist_hbm, histv)
    _bucket_offsets(histv, starts, sizes, wid, False)
    pltpu.sync_copy(u_hbm, uvv)
    u0 = uvv[pl.ds(0, 16)]
    u1 = uvv[pl.ds(16, 16)]
    iota = lax.iota(jnp.int32, 16)
    zf = jnp.zeros((16,), jnp.float32)
    bigv = jnp.full((16,), BIG, jnp.float32)

    for bi in range(2):
        b = wid + bi * 32
        bvec = jnp.full((16,), b, jnp.int32)
        sb = plsc.load_gather(starts, [bvec])[0]
        tb = plsc.load_gather(sizes, [bvec])[0]

        def init(i, c):
            accS[pl.ds(i * 16, 16)] = zf
            accQ[pl.ds(i * 16, 16)] = zf
            accN[pl.ds(i * 16, 16)] = bigv
            accX[pl.ds(i * 16, 16)] = -bigv
            return c

        lax.fori_loop(0, _ACC // 16, init, 0)

        def chunk(ci, c):
            cbase = pl.multiple_of(sb + ci * CH, 16)
            pltpu.sync_copy(srcB.at[pl.ds(cbase, CH)], srcv)
            pltpu.sync_copy(dstB.at[pl.ds(cbase, CH)], dstv)
            pltpu.sync_copy(aB.at[pl.ds(cbase, CH)], av)

            def san(g, cc):
                valid = (ci * CH + g * 16 + iota) < tb
                sv = srcv[pl.ds(g * 16, 16)]
                srcv[pl.ds(g * 16, 16)] = jnp.where(valid, sv, 0)
                dv = dstv[pl.ds(g * 16, 16)]
                dstv[pl.ds(g * 16, 16)] = jnp.where(valid, dv, NPB)
                avv = av[pl.ds(g * 16, 16)]
                av[pl.ds(g * 16, 16)] = jnp.where(valid, avv, 0.0)
                return cc

            lax.fori_loop(0, CH // 16, san, 0)
            pltpu.async_copy(q_hbm.at[srcv], rows, sem).wait()

            def grp(g, cc):
                dvec = dstv[pl.ds(g * 16, 16)]
                avec = av[pl.ds(g * 16, 16)]
                for j in range(16):
                    dloc = dvec[j]
                    aj = avec[j]
                    off = dloc * 32
                    eL = g * 16 + j
                    q0 = rows[eL, 0:16]
                    q1 = rows[eL, 16:32]
                    r0 = q0 + aj * u0
                    r1 = q1 + aj * u1
                    s0 = accS[pl.ds(off, 16)]
                    accS[pl.ds(off, 16)] = s0 + r0
                    s1 = accS[pl.ds(off + 16, 16)]
                    accS[pl.ds(off + 16, 16)] = s1 + r1
                    t0 = accQ[pl.ds(off, 16)]
                    accQ[pl.ds(off, 16)] = t0 + r0 * r0
                    t1 = accQ[pl.ds(off + 16, 16)]
                    accQ[pl.ds(off + 16, 16)] = t1 + r1 * r1
                    n0 = accN[pl.ds(off, 16)]
                    accN[pl.ds(off, 16)] = jnp.minimum(n0, r0)
                    n1 = accN[pl.ds(off + 16, 16)]
                    accN[pl.ds(off + 16, 16)] = jnp.minimum(n1, r1)
                    x0 = accX[pl.ds(off, 16)]
                    accX[pl.ds(off, 16)] = jnp.maximum(x0, r0)
                    x1 = accX[pl.ds(off + 16, 16)]
                    accX[pl.ds(off + 16, 16)] = jnp.maximum(x1, r1)
                return cc

            lax.fori_loop(0, CH // 16, grp, 0)
            return c

        nch = (tb + (CH - 1)) >> 7
        lax.fori_loop(0, nch, chunk, 0)

        wout = NPB * 32
        obase = pl.multiple_of(b * wout, 16)
        pltpu.sync_copy(accS.at[pl.ds(0, wout)], sum_o.at[pl.ds(obase, wout)])
        pltpu.sync_copy(accQ.at[pl.ds(0, wout)], sq_o.at[pl.ds(obase, wout)])
        pltpu.sync_copy(accN.at[pl.ds(0, wout)], mn_o.at[pl.ds(obase, wout)])
        pltpu.sync_copy(accX.at[pl.ds(0, wout)], mx_o.at[pl.ds(obase, wout)])


def _dot(a, b):
    return jnp.dot(a, b, precision=lax.Precision.HIGHEST,
                   preferred_element_type=jnp.float32)


def _tcpre1_body(x_ref, wb_ref, q_ref):
    onehot = (lax.broadcasted_iota(jnp.int32, (1, 32), 1) == 25).astype(jnp.float32)
    q_ref[...] = _dot(x_ref[...], wb_ref[...]) + onehot


def _tcpre2_body(o_ref, st_ref, g_ref, be_ref, wb_ref, h_ref, q_ref):
    mu = st_ref[0:1, :]
    var = st_ref[1:2, :]
    h = jax.nn.relu((o_ref[...] - mu) * lax.rsqrt(var + 1e-5) * g_ref[...]
                    + be_ref[...])
    h_ref[...] = h
    onehot = (lax.broadcasted_iota(jnp.int32, (1, 32), 1) == 25).astype(jnp.float32)
    q_ref[...] = _dot(h, wb_ref[...]) + onehot


def _tcpost_body(h_ref, sum_ref, sq_ref, mn_ref, mx_ref, wa_ref, vb_ref,
                 wf1_ref, wf2_ref, wf3_ref, bias_ref, out_ref, st_ref, scr):
    i = pl.program_id(0)
    h = h_ref[...]
    s = _dot(h, wa_ref[...]) + vb_ref[...]
    cnt = sum_ref[:, 25:26]
    cnt_c = jnp.maximum(cnt, 1.0)
    has = cnt > 0.0
    sums = sum_ref[:, 0:25]
    sqs = sq_ref[:, 0:25]
    mean = jnp.where(has, s + sums / cnt_c, 0.0)
    mn = jnp.where(has, s + mn_ref[:, 0:25], 0.0)
    mx = jnp.where(has, s + mx_ref[:, 0:25], 0.0)
    var = sqs / cnt_c - (sums / cnt_c) ** 2
    std = jnp.sqrt(jax.nn.relu(var) + 1e-5)
    x_cat = jnp.concatenate([h, mean, mn, mx, std], axis=1)
    y_cat = jnp.concatenate([mean, mn, mx, std], axis=1)
    lg = jnp.log(cnt_c + 1.0)
    o = (_dot(x_cat, wf1_ref[...]) + (lg / AVG_LOG) * _dot(y_cat, wf2_ref[...])
         + (AVG_LOG / lg) * _dot(y_cat, wf3_ref[...]) + bias_ref[...])
    out_ref[...] = o

    @pl.when(i == 0)
    def _():
        scr[...] = jnp.zeros_like(scr)

    scr[0, 0:5] += jnp.sum(o, axis=0)
    scr[1, 0:5] += jnp.sum(o * o, axis=0)

    @pl.when(i == NBLK - 1)
    def _():
        mu = scr[0:1, 0:5] / N
        ex2 = scr[1:2, 0:5] / N
        st_ref[...] = jnp.concatenate([mu, ex2 - mu * mu], axis=0)


def _tcfinal_body(o_ref, st_ref, g_ref, be_ref, batch_ref,
                  w1_ref, b1_ref, w2_ref, b2_ref, w3_ref, b3_ref,
                  out_ref, scr):
    i = pl.program_id(0)
    mu = st_ref[0:1, :]
    var = st_ref[1:2, :]
    h = jax.nn.relu((o_ref[...] - mu) * lax.rsqrt(var + 1e-5) * g_ref[...]
                    + be_ref[...])
    seg = batch_ref[0, 0, :]
    onehot = (seg[:, None] == lax.broadcasted_iota(jnp.int32, (BLK, NG), 1)
              ).astype(jnp.float32)
    pooled = lax.dot_general(onehot, h, (((0,), (0,)), ((), ())),
                             precision=lax.Precision.HIGHEST,
                             preferred_element_type=jnp.float32)

    @pl.when(i == 0)
    def _():
        scr[...] = jnp.zeros_like(scr)

    scr[:, 0:5] += pooled

    @pl.when(i == NBLK - 1)
    def _():
        p = scr[:, 0:5]
        z1 = jax.nn.relu(_dot(p, w1_ref[...]) + b1_ref[...])
        z2 = jax.nn.relu(_dot(z1, w2_ref[...]) + b2_ref[...])
        out_ref[...] = _dot(z2, w3_ref[...]) + b3_ref[...]


def _row_spec(cols):
    return pl.BlockSpec((BLK, cols), lambda i: (i, 0))


def _full_spec(shape):
    nd = len(shape)
    return pl.BlockSpec(shape, lambda i: (0,) * nd)


def kernel(x, edge_index, edge_attr, batch, Wpre, bpre, Wedge, bedge, Wpost,
           bpost, Wlin, blin, bn_gamma, bn_beta, W1, b1, W2, b2, W3, b3):
    f32 = jnp.float32
    src = edge_index[0]
    dst = edge_index[1]
    a = edge_attr[:, 0]
    npad = EPAD - E
    srcp = jnp.concatenate([src, jnp.zeros((npad,), jnp.int32)])
    dstp = jnp.concatenate([dst, jnp.full((npad,), NPAD - 1, jnp.int32)])
    ap = jnp.concatenate([a, jnp.zeros((npad,), f32)])

    hist = _k_hist(dstp)
    srcB, dstB, aB = _k_scatter(srcp, dstp, ap, hist)

    eye_mask = jnp.asarray(np.kron(np.eye(T), np.ones((F, 1))), f32)

    def fold(ws):
        return ws.reshape(TF, 1) * eye_mask

    def layer_weights(l):
        Wp = Wpre[l]
        WA = Wp[:, 0:F, :].transpose(1, 0, 2).reshape(F, TF)
        WB = Wp[:, F:2 * F, :].transpose(1, 0, 2).reshape(F, TF)
        WC = Wp[:, 2 * F:3 * F, :].transpose(1, 0, 2).reshape(F, TF)
        u = Wedge[l][0] @ WC
        vb = bedge[l] @ WC + bpre[l].reshape(TF)
        upad = jnp.concatenate([u, jnp.zeros((7,), f32)])
        WBpad = jnp.concatenate([WB, jnp.zeros((F, 7), f32)], axis=1)
        Wp2 = Wpost[l][:, :, 0]
        wh = Wp2[:, 0:F]
        folds = [fold(Wp2[:, F + k * F:F + (k + 1) * F]) for k in range(12)]
        Wf1 = jnp.concatenate([wh.T] + folds[0:4], axis=0) @ Wlin[l]
        Wf2 = jnp.concatenate(folds[4:8], axis=0) @ Wlin[l]
        Wf3 = jnp.concatenate(folds[8:12], axis=0) @ Wlin[l]
        bias = (bpost[l][:, 0] @ Wlin[l] + blin[l]).reshape(1, F)
        return WA, WBpad, upad, vb.reshape(1, TF), Wf1, Wf2, Wf3, bias

    def run_edge_phase(qpad, upad):
        outs = _k_edge(srcB, dstB, aB, hist, upad, qpad)
        return [o.reshape(NPAD, 32)[:N] for o in outs]

    def tcpost(h, stats4, WA, vb, Wf1, Wf2, Wf3, bias):
        s_, q_, n_, x_ = stats4
        return pl.pallas_call(
            _tcpost_body,
            grid=(NBLK,),
            in_specs=[_row_spec(5), _row_spec(32), _row_spec(32),
                      _row_spec(32), _row_spec(32), _full_spec((F, TF)),
                      _full_spec((1, TF)), _full_spec((105, 5)),
                      _full_spec((100, 5)), _full_spec((100, 5)),
                      _full_spec((1, 5))],
            out_specs=[_row_spec(5), _full_spec((2, 5))],
            out_shape=[jax.ShapeDtypeStruct((N, 5), f32),
                       jax.ShapeDtypeStruct((2, 5), f32)],
            scratch_shapes=[pltpu.VMEM((8, 128), f32)],
        )(h, s_, q_, n_, x_, WA, vb, Wf1, Wf2, Wf3, bias)

    WA1, WBpad1, upad1, vb1, Wf11, Wf21, Wf31, bias1 = layer_weights(0)
    qpad1 = pl.pallas_call(
        _tcpre1_body,
        grid=(NBLK,),
        in_specs=[_row_spec(5), _full_spec((F, 32))],
        out_specs=_row_spec(32),
        out_shape=jax.ShapeDtypeStruct((N, 32), f32),
    )(x, WBpad1)
    st4_1 = run_edge_phase(qpad1, upad1)
    out1, bstats1 = tcpost(x, st4_1, WA1, vb1, Wf11, Wf21, Wf31, bias1)

    WA2, WBpad2, upad2, vb2, Wf12, Wf22, Wf32, bias2 = layer_weights(1)
    h2, qpad2 = pl.pallas_call(
        _tcpre2_body,
        grid=(NBLK,),
        in_specs=[_row_spec(5), _full_spec((2, 5)), _full_spec((1, 5)),
                  _full_spec((1, 5)), _full_spec((F, 32))],
        out_specs=[_row_spec(5), _row_spec(32)],
        out_shape=[jax.ShapeDtypeStruct((N, 5), f32),
                   jax.ShapeDtypeStruct((N, 32), f32)],
    )(out1, bstats1, bn_gamma[0].reshape(1, 5), bn_beta[0].reshape(1, 5),
      WBpad2)
    st4_2 = run_edge_phase(qpad2, upad2)
    out2, bstats2 = tcpost(h2, st4_2, WA2, vb2, Wf12, Wf22, Wf32, bias2)

    batch3d = batch.reshape(NBLK, 1, BLK)
    z = pl.pallas_call(
        _tcfinal_body,
        grid=(NBLK,),
        in_specs=[_row_spec(5), _full_spec((2, 5)), _full_spec((1, 5)),
                  _full_spec((1, 5)),
                  pl.BlockSpec((1, 1, BLK), lambda i: (i, 0, 0)),
                  _full_spec((5, 5)), _full_spec((1, 5)),
                  _full_spec((5, 10)), _full_spec((1, 10)),
                  _full_spec((10, 10)), _full_spec((1, 10))],
        out_specs=_full_spec((NG, 10)),
        out_shape=jax.ShapeDtypeStruct((NG, 10), f32),
        scratch_shapes=[pltpu.VMEM((NG, 128), f32)],
    )(out2, bstats2, bn_gamma[1].reshape(1, 5), bn_beta[1].reshape(1, 5),
      batch3d, W1, b1.reshape(1, 5), W2, b2.reshape(1, 10), W3,
      b3.reshape(1, 10))
    return z


# speedup vs baseline: 124.8898x; 1.1275x over previous
import functools
import math

import jax
import jax.numpy as jnp
import numpy as np
from jax import lax
from jax.experimental import pallas as pl
from jax.experimental.pallas import tpu as pltpu, tpu_sc as plsc

N = 50000
E = 800000
NG = 512
F = 5
T = 5
TF = T * F

_DEG = np.array([0, 0, 0, 0, 0, 0, 200, 400, 800, 1200, 1800, 2400, 3000,
                 3600, 4000, 4300, 4400, 4400, 4300, 4000, 3600, 3000, 2400,
                 1800, 1200, 800, 400, 200], dtype=np.float64)
AVG_LOG = float((np.log(np.arange(_DEG.shape[0]) + 1.0) * _DEG).sum() / _DEG.sum())

NB = 128
NPB = 392
NPAD = NB * NPB
EPT = 25600
EPAD = 32 * EPT
CAP = 821760
CH = 512
BLK = 2000
NBLK = 25
BIG = 3.0e38

_mesh = plsc.VectorSubcoreMesh(core_axis_name="c", subcore_axis_name="s")
_sc_params = pltpu.CompilerParams(
    needs_layout_passes=False, use_tc_tiling_on_sc=False)


def _wid():
    return lax.axis_index("s") * 2 + lax.axis_index("c")


def _bucket_of(d):
    return ((d >> 3) * 2675) >> 17


def _bucket_offsets(histv, offs, sizes, wid, with_pre):
    carry = jnp.int32(0)
    zero16 = jnp.zeros((16,), jnp.int32)
    for g in range(NB // 16):
        tot = zero16
        pre = zero16
        for t in range(32):
            hrow = histv[t, g * 16:(g + 1) * 16]
            tot = tot + hrow
            if with_pre:
                pre = pre + jnp.where(t < wid, hrow, zero16)
        padded = (tot + 15) & (-16)
        incl = plsc.cumsum(padded)
        excl = incl - padded + carry
        if with_pre:
            offs[pl.ds(g * 16, 16)] = excl + pre
        else:
            offs[pl.ds(g * 16, 16)] = excl
        if sizes is not None:
            sizes[pl.ds(g * 16, 16)] = tot
        carry = carry + incl[15]


@functools.partial(
    pl.kernel,
    mesh=_mesh,
    compiler_params=_sc_params,
    out_type=jax.ShapeDtypeStruct((32, NB), jnp.int32),
    scratch_types=[
        pltpu.VMEM((512,), jnp.int32),
        pltpu.VMEM((NB,), jnp.int32),
    ],
)
def _k_hist(dst_hbm, out_hbm, dstv, hist):
    wid = _wid()

    def z(i, c):
        hist[pl.ds(i * 16, 16)] = jnp.zeros((16,), jnp.int32)
        return c

    lax.fori_loop(0, NB // 16, z, 0)
    iota = lax.iota(jnp.int32, 16)

    def chunk(ci, c):
        pltpu.sync_copy(dst_hbm.at[pl.ds(wid * EPT + ci * 512, 512)], dstv)

        def grp(g, cc):
            d = dstv[pl.ds(g * 16, 16)]
            b = _bucket_of(d)
            cnt = jnp.zeros((16,), jnp.int32)
            rank = jnp.zeros((16,), jnp.int32)
            for j in range(16):
                eq = b == b[j]
                cnt = cnt + jnp.where(eq, 1, 0)
                rank = rank + jnp.where(eq & (iota > j), 1, 0)
            uniq = rank == 0
            old = plsc.load_gather(hist, [b], mask=uniq)
            plsc.store_scatter(hist, [b], old + cnt, mask=uniq)
            return cc

        lax.fori_loop(0, 32, grp, 0)
        return c

    lax.fori_loop(0, EPT // 512, chunk, 0)
    pltpu.sync_copy(hist, out_hbm.at[wid])


@functools.partial(
    pl.kernel,
    mesh=_mesh,
    compiler_params=_sc_params,
    out_type=[
        jax.ShapeDtypeStruct((CAP,), jnp.int32),
        jax.ShapeDtypeStruct((CAP,), jnp.int32),
        jax.ShapeDtypeStruct((CAP,), jnp.float32),
    ],
    scratch_types=[
        pltpu.VMEM((32, NB), jnp.int32),
        pltpu.VMEM((NB,), jnp.int32),
        pltpu.VMEM((CH,), jnp.int32),
        pltpu.VMEM((CH,), jnp.int32),
        pltpu.VMEM((CH,), jnp.float32),
        pltpu.VMEM((CH // 128, 128), jnp.int32),
        pltpu.VMEM((CH,), jnp.int32),
        pltpu.SemaphoreType.DMA,
        pltpu.SemaphoreType.DMA,
    ],
)
def _k_scatter(src_hbm, dst_hbm, a_hbm, hist_hbm, srcB, dstB, aB,
               histv, offs, srcv, dstv, av, posv, dlv, sem_st, sem_sc):
    wid = _wid()
    pltpu.sync_copy(hist_hbm, histv)
    _bucket_offsets(histv, offs, None, wid, True)
    iota = lax.iota(jnp.int32, 16)

    def chunk(ci, c):
        base = wid * EPT + ci * CH
        s1 = pltpu.async_copy(src_hbm.at[pl.ds(base, CH)], srcv, sem_st)
        s2 = pltpu.async_copy(dst_hbm.at[pl.ds(base, CH)], dstv, sem_st)
        s3 = pltpu.async_copy(a_hbm.at[pl.ds(base, CH)], av, sem_st)
        s1.wait()
        s2.wait()
        s3.wait()
        for g in range(CH // 16):
            d = dstv[pl.ds(g * 16, 16)]
            b = _bucket_of(d)
            cnt = jnp.zeros((16,), jnp.int32)
            rank = jnp.zeros((16,), jnp.int32)
            for j in range(16):
                eq = b == b[j]
                cnt = cnt + jnp.where(eq, 1, 0)
                rank = rank + jnp.where(eq & (iota > j), 1, 0)
            uniq = rank == 0
            basev = plsc.load_gather(offs, [b])
            plsc.store_scatter(offs, [b], basev + cnt, mask=uniq)
            posv[g // 8, pl.ds((g % 8) * 16, 16)] = basev + rank
            dlv[pl.ds(g * 16, 16)] = d - b * NPB
        copies = []
        for j in range(CH // 128):
            sl = pl.ds(j * 128, 128)
            copies.append(pltpu.async_copy(srcv.at[sl], srcB.at[posv.at[j]], sem_sc))
            copies.append(pltpu.async_copy(dlv.at[sl], dstB.at[posv.at[j]], sem_sc))
            copies.append(pltpu.async_copy(av.at[sl], aB.at[posv.at[j]], sem_sc))
        for cp in copies:
            cp.wait()
        return c

    lax.fori_loop(0, EPT // CH, chunk, 0)


_ACC = (NPB + 1) * 32
_BPT = NB // 32


@functools.partial(
    pl.kernel,
    mesh=_mesh,
    compiler_params=_sc_params,
    out_type=[jax.ShapeDtypeStruct((NPAD * 32,), jnp.float32)
              for _ in range(4)],
    scratch_types=[
        pltpu.VMEM((32, NB), jnp.int32),
        pltpu.VMEM((NB,), jnp.int32),
        pltpu.VMEM((NB,), jnp.int32),
        pltpu.VMEM((CH,), jnp.int32),
        pltpu.VMEM((CH,), jnp.int32),
        pltpu.VMEM((CH,), jnp.float32),
        pltpu.VMEM((CH, 32), jnp.float32),
        pltpu.VMEM((CH,), jnp.int32),
        pltpu.VMEM((CH,), jnp.int32),
        pltpu.VMEM((CH,), jnp.float32),
        pltpu.VMEM((CH, 32), jnp.float32),
        pltpu.VMEM((32,), jnp.float32),
        pltpu.VMEM((_ACC,), jnp.float32),
        pltpu.VMEM((_ACC,), jnp.float32),
        pltpu.VMEM((_ACC,), jnp.float32),
        pltpu.VMEM((_ACC,), jnp.float32),
        pltpu.SemaphoreType.DMA,
        pltpu.SemaphoreType.DMA,
        pltpu.SemaphoreType.DMA,
        pltpu.SemaphoreType.DMA,
    ],
)
def _k_edge(srcB, dstB, aB, hist_hbm, u_hbm, q_hbm,
            sum_o, sq_o, mn_o, mx_o,
            histv, starts, sizes, srcv0, dstv0, av0, rows0,
            srcv1, dstv1, av1, rows1, uvv,
            accS, accQ, accN, accX, st0, st1, sg0, sg1):
    wid = _wid()
    pltpu.sync_copy(hist_hbm, histv)
    _bucket_offsets(histv, starts, sizes, wid, False)
    pltpu.sync_copy(u_hbm, uvv)
    u0 = uvv[pl.ds(0, 16)]
    u1 = uvv[pl.ds(16, 16)]
    iota = lax.iota(jnp.int32, 16)
    zf = jnp.zeros((16,), jnp.float32)
    bigv = jnp.full((16,), BIG, jnp.float32)
    slots = ((srcv0, dstv0, av0, rows0, st0, sg0),
             (srcv1, dstv1, av1, rows1, st1, sg1))

    for bi in range(_BPT):
        b = wid + bi * 32
        bvec = jnp.full((16,), b, jnp.int32)
        sb = plsc.load_gather(starts, [bvec])[0]
        tb = plsc.load_gather(sizes, [bvec])[0]
        nch = (tb + (CH - 1)) >> 9

        def init(i, c):
            accS[pl.ds(i * 16, 16)] = zf
            accQ[pl.ds(i * 16, 16)] = zf
            accN[pl.ds(i * 16, 16)] = bigv
            accX[pl.ds(i * 16, 16)] = -bigv
            return c

        lax.fori_loop(0, _ACC // 16, init, 0)

        def issue_stage(ci, slot):
            srcv, dstv, av, _, st, _ = slots[slot]

            @pl.when(ci < nch)
            def _():
                cbase = pl.multiple_of(sb + ci * CH, 16)
                pltpu.async_copy(srcB.at[pl.ds(cbase, CH)], srcv, st)
                pltpu.async_copy(dstB.at[pl.ds(cbase, CH)], dstv, st)
                pltpu.async_copy(aB.at[pl.ds(cbase, CH)], av, st)

        def run_chunk(ci, slot, nxt_ci, nxt_slot):
            srcv, dstv, av, rows, st, sg = slots[slot]

            @pl.when(ci < nch)
            def _():
                cbase = pl.multiple_of(sb + ci * CH, 16)
                pltpu.make_async_copy(srcB.at[pl.ds(cbase, CH)], srcv, st).wait()
                pltpu.make_async_copy(dstB.at[pl.ds(cbase, CH)], dstv, st).wait()
                pltpu.make_async_copy(aB.at[pl.ds(cbase, CH)], av, st).wait()

                def san(g, cc):
                    valid = (ci * CH + g * 16 + iota) < tb
                    sv = srcv[pl.ds(g * 16, 16)]
                    srcv[pl.ds(g * 16, 16)] = jnp.where(valid, sv, 0)
                    dv = dstv[pl.ds(g * 16, 16)]
                    dstv[pl.ds(g * 16, 16)] = jnp.where(valid, dv, NPB)
                    avv = av[pl.ds(g * 16, 16)]
                    av[pl.ds(g * 16, 16)] = jnp.where(valid, avv, 0.0)
                    return cc

                lax.fori_loop(0, CH // 16, san, 0)
                gathers = []
                for k in range(CH // 128):
                    gathers.append(pltpu.async_copy(
                        q_hbm.at[srcv.at[pl.ds(k * 128, 128)]],
                        rows.at[pl.ds(k * 128, 128)], sg))
                issue_stage(nxt_ci, nxt_slot)
                for gcp in gathers:
                    gcp.wait()

                def grp(g, cc):
                    dvec = dstv[pl.ds(g * 16, 16)]
                    avec = av[pl.ds(g * 16, 16)]
                    for j in range(16):
                        dloc = dvec[j]
                        aj = avec[j]
                        off = dloc * 32
                        eL = g * 16 + j
                        q0 = rows[eL, 0:16]
                        q1 = rows[eL, 16:32]
                        r0 = q0 + aj * u0
                        r1 = q1 + aj * u1
                        s0 = accS[pl.ds(off, 16)]
                        accS[pl.ds(off, 16)] = s0 + r0
                        s1 = accS[pl.ds(off + 16, 16)]
                        accS[pl.ds(off + 16, 16)] = s1 + r1
                        t0 = accQ[pl.ds(off, 16)]
                        accQ[pl.ds(off, 16)] = t0 + r0 * r0
                        t1 = accQ[pl.ds(off + 16, 16)]
                        accQ[pl.ds(off + 16, 16)] = t1 + r1 * r1
                        n0 = accN[pl.ds(off, 16)]
                        accN[pl.ds(off, 16)] = jnp.minimum(n0, r0)
                        n1 = accN[pl.ds(off + 16, 16)]
                        accN[pl.ds(off + 16, 16)] = jnp.minimum(n1, r1)
                        x0 = accX[pl.ds(off, 16)]
                        accX[pl.ds(off, 16)] = jnp.maximum(x0, r0)
                        x1 = accX[pl.ds(off + 16, 16)]
                        accX[pl.ds(off + 16, 16)] = jnp.maximum(x1, r1)
                    return cc

                lax.fori_loop(0, CH // 16, grp, 0)

        issue_stage(jnp.int32(0), 0)

        def pair(k, c):
            ci0 = k * 2
            run_chunk(ci0, 0, ci0 + 1, 1)
            run_chunk(ci0 + 1, 1, ci0 + 2, 0)
            return c

        lax.fori_loop(0, (nch + 1) >> 1, pair, 0)

        wout = NPB * 32
        obase = pl.multiple_of(b * wout, 16)
        pltpu.sync_copy(accS.at[pl.ds(0, wout)], sum_o.at[pl.ds(obase, wout)])
        pltpu.sync_copy(accQ.at[pl.ds(0, wout)], sq_o.at[pl.ds(obase, wout)])
        pltpu.sync_copy(accN.at[pl.ds(0, wout)], mn_o.at[pl.ds(obase, wout)])
        pltpu.sync_copy(accX.at[pl.ds(0, wout)], mx_o.at[pl.ds(obase, wout)])


def _dot(a, b):
    return jnp.dot(a, b, precision=lax.Precision.HIGHEST,
                   preferred_element_type=jnp.float32)


def _tcpre1_body(x_ref, wb_ref, q_ref):
    onehot = (lax.broadcasted_iota(jnp.int32, (1, 32), 1) == 25).astype(jnp.float32)
    q_ref[...] = _dot(x_ref[...], wb_ref[...]) + onehot


def _tcpre2_body(o_ref, st_ref, g_ref, be_ref, wb_ref, h_ref, q_ref):
    mu = st_ref[0:1, :]
    var = st_ref[1:2, :]
    h = jax.nn.relu((o_ref[...] - mu) * lax.rsqrt(var + 1e-5) * g_ref[...]
                    + be_ref[...])
    h_ref[...] = h
    onehot = (lax.broadcasted_iota(jnp.int32, (1, 32), 1) == 25).astype(jnp.float32)
    q_ref[...] = _dot(h, wb_ref[...]) + onehot


def _tcpost_body(h_ref, sum_ref, sq_ref, mn_ref, mx_ref, wa_ref, vb_ref,
                 wf1_ref, wf2_ref, wf3_ref, bias_ref, out_ref, st_ref, scr):
    i = pl.program_id(0)
    h = h_ref[...]
    s = _dot(h, wa_ref[...]) + vb_ref[...]
    cnt = sum_ref[:, 25:26]
    cnt_c = jnp.maximum(cnt, 1.0)
    has = cnt > 0.0
    sums = sum_ref[:, 0:25]
    sqs = sq_ref[:, 0:25]
    mean = jnp.where(has, s + sums / cnt_c, 0.0)
    mn = jnp.where(has, s + mn_ref[:, 0:25], 0.0)
    mx = jnp.where(has, s + mx_ref[:, 0:25], 0.0)
    var = sqs / cnt_c - (sums / cnt_c) ** 2
    std = jnp.sqrt(jax.nn.relu(var) + 1e-5)
    x_cat = jnp.concatenate([h, mean, mn, mx, std], axis=1)
    y_cat = jnp.concatenate([mean, mn, mx, std], axis=1)
    lg = jnp.log(cnt_c + 1.0)
    o = (_dot(x_cat, wf1_ref[...]) + (lg / AVG_LOG) * _dot(y_cat, wf2_ref[...])
         + (AVG_LOG / lg) * _dot(y_cat, wf3_ref[...]) + bias_ref[...])
    out_ref[...] = o

    @pl.when(i == 0)
    def _():
        scr[...] = jnp.zeros_like(scr)

    scr[0, 0:5] += jnp.sum(o, axis=0)
    scr[1, 0:5] += jnp.sum(o * o, axis=0)

    @pl.when(i == NBLK - 1)
    def _():
        mu = scr[0:1, 0:5] / N
        ex2 = scr[1:2, 0:5] / N
        st_ref[...] = jnp.concatenate([mu, ex2 - mu * mu], axis=0)


def _tcfinal_body(o_ref, st_ref, g_ref, be_ref, batch_ref,
                  w1_ref, b1_ref, w2_ref, b2_ref, w3_ref, b3_ref,
                  out_ref, scr):
    i = pl.program_id(0)
    mu = st_ref[0:1, :]
    var = st_ref[1:2, :]
    h = jax.nn.relu((o_ref[...] - mu) * lax.rsqrt(var + 1e-5) * g_ref[...]
                    + be_ref[...])
    seg = batch_ref[0, 0, :]
    onehot = (seg[:, None] == lax.broadcasted_iota(jnp.int32, (BLK, NG), 1)
              ).astype(jnp.float32)
    pooled = lax.dot_general(onehot, h, (((0,), (0,)), ((), ())),
                             precision=lax.Precision.HIGHEST,
                             preferred_element_type=jnp.float32)

    @pl.when(i == 0)
    def _():
        scr[...] = jnp.zeros_like(scr)

    scr[:, 0:5] += pooled

    @pl.when(i == NBLK - 1)
    def _():
        p = scr[:, 0:5]
        z1 = jax.nn.relu(_dot(p, w1_ref[...]) + b1_ref[...])
        z2 = jax.nn.relu(_dot(z1, w2_ref[...]) + b2_ref[...])
        out_ref[...] = _dot(z2, w3_ref[...]) + b3_ref[...]


def _row_spec(cols):
    return pl.BlockSpec((BLK, cols), lambda i: (i, 0))


def _full_spec(shape):
    nd = len(shape)
    return pl.BlockSpec(shape, lambda i: (0,) * nd)


def kernel(x, edge_index, edge_attr, batch, Wpre, bpre, Wedge, bedge, Wpost,
           bpost, Wlin, blin, bn_gamma, bn_beta, W1, b1, W2, b2, W3, b3):
    f32 = jnp.float32
    src = edge_index[0]
    dst = edge_index[1]
    a = edge_attr[:, 0]
    npad = EPAD - E
    srcp = jnp.concatenate([src, jnp.zeros((npad,), jnp.int32)])
    dstp = jnp.concatenate([dst, jnp.full((npad,), NPAD - 1, jnp.int32)])
    ap = jnp.concatenate([a, jnp.zeros((npad,), f32)])

    hist = _k_hist(dstp)
    srcB, dstB, aB = _k_scatter(srcp, dstp, ap, hist)

    eye_mask = jnp.asarray(np.kron(np.eye(T), np.ones((F, 1))), f32)

    def fold(ws):
        return ws.reshape(TF, 1) * eye_mask

    def layer_weights(l):
        Wp = Wpre[l]
        WA = Wp[:, 0:F, :].transpose(1, 0, 2).reshape(F, TF)
        WB = Wp[:, F:2 * F, :].transpose(1, 0, 2).reshape(F, TF)
        WC = Wp[:, 2 * F:3 * F, :].transpose(1, 0, 2).reshape(F, TF)
        u = Wedge[l][0] @ WC
        vb = bedge[l] @ WC + bpre[l].reshape(TF)
        upad = jnp.concatenate([u, jnp.zeros((7,), f32)])
        WBpad = jnp.concatenate([WB, jnp.zeros((F, 7), f32)], axis=1)
        Wp2 = Wpost[l][:, :, 0]
        wh = Wp2[:, 0:F]
        folds = [fold(Wp2[:, F + k * F:F + (k + 1) * F]) for k in range(12)]
        Wf1 = jnp.concatenate([wh.T] + folds[0:4], axis=0) @ Wlin[l]
        Wf2 = jnp.concatenate(folds[4:8], axis=0) @ Wlin[l]
        Wf3 = jnp.concatenate(folds[8:12], axis=0) @ Wlin[l]
        bias = (bpost[l][:, 0] @ Wlin[l] + blin[l]).reshape(1, F)
        return WA, WBpad, upad, vb.reshape(1, TF), Wf1, Wf2, Wf3, bias

    def run_edge_phase(qpad, upad):
        outs = _k_edge(srcB, dstB, aB, hist, upad, qpad)
        return [o.reshape(NPAD, 32)[:N] for o in outs]

    def tcpost(h, stats4, WA, vb, Wf1, Wf2, Wf3, bias):
        s_, q_, n_, x_ = stats4
        return pl.pallas_call(
            _tcpost_body,
            grid=(NBLK,),
            in_specs=[_row_spec(5), _row_spec(32), _row_spec(32),
                      _row_spec(32), _row_spec(32), _full_spec((F, TF)),
                      _full_spec((1, TF)), _full_spec((105, 5)),
                      _full_spec((100, 5)), _full_spec((100, 5)),
                      _full_spec((1, 5))],
            out_specs=[_row_spec(5), _full_spec((2, 5))],
            out_shape=[jax.ShapeDtypeStruct((N, 5), f32),
                       jax.ShapeDtypeStruct((2, 5), f32)],
            scratch_shapes=[pltpu.VMEM((8, 128), f32)],
        )(h, s_, q_, n_, x_, WA, vb, Wf1, Wf2, Wf3, bias)

    WA1, WBpad1, upad1, vb1, Wf11, Wf21, Wf31, bias1 = layer_weights(0)
    qpad1 = pl.pallas_call(
        _tcpre1_body,
        grid=(NBLK,),
        in_specs=[_row_spec(5), _full_spec((F, 32))],
        out_specs=_row_spec(32),
        out_shape=jax.ShapeDtypeStruct((N, 32), f32),
    )(x, WBpad1)
    st4_1 = run_edge_phase(qpad1, upad1)
    out1, bstats1 = tcpost(x, st4_1, WA1, vb1, Wf11, Wf21, Wf31, bias1)

    WA2, WBpad2, upad2, vb2, Wf12, Wf22, Wf32, bias2 = layer_weights(1)
    h2, qpad2 = pl.pallas_call(
        _tcpre2_body,
        grid=(NBLK,),
        in_specs=[_row_spec(5), _full_spec((2, 5)), _full_spec((1, 5)),
                  _full_spec((1, 5)), _full_spec((F, 32))],
        out_specs=[_row_spec(5), _row_spec(32)],
        out_shape=[jax.ShapeDtypeStruct((N, 5), f32),
                   jax.ShapeDtypeStruct((N, 32), f32)],
    )(out1, bstats1, bn_gamma[0].reshape(1, 5), bn_beta[0].reshape(1, 5),
      WBpad2)
    st4_2 = run_edge_phase(qpad2, upad2)
    out2, bstats2 = tcpost(h2, st4_2, WA2, vb2, Wf12, Wf22, Wf32, bias2)

    batch3d = batch.reshape(NBLK, 1, BLK)
    z = pl.pallas_call(
        _tcfinal_body,
        grid=(NBLK,),
        in_specs=[_row_spec(5), _full_spec((2, 5)), _full_spec((1, 5)),
                  _full_spec((1, 5)),
                  pl.BlockSpec((1, 1, BLK), lambda i: (i, 0, 0)),
                  _full_spec((5, 5)), _full_spec((1, 5)),
                  _full_spec((5, 10)), _full_spec((1, 10)),
                  _full_spec((10, 10)), _full_spec((1, 10))],
        out_specs=_full_spec((NG, 10)),
        out_shape=jax.ShapeDtypeStruct((NG, 10), f32),
        scratch_shapes=[pltpu.VMEM((NG, 128), f32)],
    )(out2, bstats2, bn_gamma[1].reshape(1, 5), bn_beta[1].reshape(1, 5),
      batch3d, W1, b1.reshape(1, 5), W2, b2.reshape(1, 10), W3,
      b3.reshape(1, 10))
    return z


# speedup vs baseline: 125.8759x; 1.0079x over previous
import functools
import math

import jax
import jax.numpy as jnp
import numpy as np
from jax import lax
from jax.experimental import pallas as pl
from jax.experimental.pallas import tpu as pltpu, tpu_sc as plsc

N = 50000
E = 800000
NG = 512
F = 5
T = 5
TF = T * F

_DEG = np.array([0, 0, 0, 0, 0, 0, 200, 400, 800, 1200, 1800, 2400, 3000,
                 3600, 4000, 4300, 4400, 4400, 4300, 4000, 3600, 3000, 2400,
                 1800, 1200, 800, 400, 200], dtype=np.float64)
AVG_LOG = float((np.log(np.arange(_DEG.shape[0]) + 1.0) * _DEG).sum() / _DEG.sum())

NB = 128
NPB = 392
NPAD = NB * NPB
EPT = 25600
EPAD = 32 * EPT
CAP = 821760
CH = 512
BLK = 2000
NBLK = 25
BIG = 3.0e38

_mesh = plsc.VectorSubcoreMesh(core_axis_name="c", subcore_axis_name="s")
_sc_params = pltpu.CompilerParams(
    needs_layout_passes=False, use_tc_tiling_on_sc=False)


def _wid():
    return lax.axis_index("s") * 2 + lax.axis_index("c")


def _bucket_of(d):
    return ((d >> 3) * 2675) >> 17


def _bucket_offsets(histv, offs, sizes, wid, with_pre):
    carry = jnp.int32(0)
    zero16 = jnp.zeros((16,), jnp.int32)
    for g in range(NB // 16):
        tot = zero16
        pre = zero16
        for t in range(32):
            hrow = histv[t, g * 16:(g + 1) * 16]
            tot = tot + hrow
            if with_pre:
                pre = pre + jnp.where(t < wid, hrow, zero16)
        padded = (tot + 15) & (-16)
        incl = plsc.cumsum(padded)
        excl = incl - padded + carry
        if with_pre:
            offs[pl.ds(g * 16, 16)] = excl + pre
        else:
            offs[pl.ds(g * 16, 16)] = excl
        if sizes is not None:
            sizes[pl.ds(g * 16, 16)] = tot
        carry = carry + incl[15]


@functools.partial(
    pl.kernel,
    mesh=_mesh,
    compiler_params=_sc_params,
    out_type=jax.ShapeDtypeStruct((32, NB), jnp.int32),
    scratch_types=[
        pltpu.VMEM((512,), jnp.int32),
        pltpu.VMEM((NB,), jnp.int32),
    ],
)
def _k_hist(dst_hbm, out_hbm, dstv, hist):
    wid = _wid()

    def z(i, c):
        hist[pl.ds(i * 16, 16)] = jnp.zeros((16,), jnp.int32)
        return c

    lax.fori_loop(0, NB // 16, z, 0)
    iota = lax.iota(jnp.int32, 16)

    def chunk(ci, c):
        pltpu.sync_copy(dst_hbm.at[pl.ds(wid * EPT + ci * 512, 512)], dstv)

        def grp(g, cc):
            d = dstv[pl.ds(g * 16, 16)]
            b = _bucket_of(d)
            cnt = jnp.zeros((16,), jnp.int32)
            rank = jnp.zeros((16,), jnp.int32)
            for j in range(16):
                eq = b == b[j]
                cnt = cnt + jnp.where(eq, 1, 0)
                rank = rank + jnp.where(eq & (iota > j), 1, 0)
            uniq = rank == 0
            old = plsc.load_gather(hist, [b], mask=uniq)
            plsc.store_scatter(hist, [b], old + cnt, mask=uniq)
            return cc

        lax.fori_loop(0, 32, grp, 0)
        return c

    lax.fori_loop(0, EPT // 512, chunk, 0)
    pltpu.sync_copy(hist, out_hbm.at[wid])


@functools.partial(
    pl.kernel,
    mesh=_mesh,
    compiler_params=_sc_params,
    out_type=[
        jax.ShapeDtypeStruct((CAP,), jnp.int32),
        jax.ShapeDtypeStruct((CAP,), jnp.int32),
        jax.ShapeDtypeStruct((CAP,), jnp.float32),
    ],
    scratch_types=[
        pltpu.VMEM((32, NB), jnp.int32),
        pltpu.VMEM((NB,), jnp.int32),
        pltpu.VMEM((CH,), jnp.int32),
        pltpu.VMEM((CH,), jnp.int32),
        pltpu.VMEM((CH,), jnp.float32),
        pltpu.VMEM((CH,), jnp.int32),
        pltpu.VMEM((CH,), jnp.int32),
        pltpu.VMEM((CH,), jnp.float32),
        pltpu.VMEM((CH // 128, 128), jnp.int32),
        pltpu.VMEM((CH // 128, 128), jnp.int32),
        pltpu.VMEM((CH,), jnp.int32),
        pltpu.VMEM((CH,), jnp.int32),
        pltpu.VMEM((CH,), jnp.int32),
        pltpu.VMEM((CH,), jnp.float32),
        pltpu.VMEM((CH,), jnp.int32),
        pltpu.VMEM((CH,), jnp.float32),
        pltpu.VMEM((128,), jnp.int32),
        pltpu.SemaphoreType.DMA,
        pltpu.SemaphoreType.DMA,
        pltpu.SemaphoreType.DMA,
        pltpu.SemaphoreType.DMA,
    ],
)
def _k_scatter(src_hbm, dst_hbm, a_hbm, hist_hbm, srcB, dstB, aB,
               histv, offs, srcv0, dstv0, av0, srcv1, dstv1, av1,
               posv0, posv1, dlv0, dlv1, srcs0, avs0, srcs1, avs1,
               dmy, st0, st1, sc0, sc1):
    wid = _wid()
    pltpu.sync_copy(hist_hbm, histv)
    _bucket_offsets(histv, offs, None, wid, True)
    iota = lax.iota(jnp.int32, 16)
    NCH = EPT // CH
    slots = ((srcv0, dstv0, av0, posv0, dlv0, srcs0, avs0, st0, sc0),
             (srcv1, dstv1, av1, posv1, dlv1, srcs1, avs1, st1, sc1))

    def issue_stage(ci, slot):
        srcv, dstv, av = slots[slot][0:3]
        st = slots[slot][7]
        base = wid * EPT + ci * CH
        pltpu.async_copy(src_hbm.at[pl.ds(base, CH)], srcv, st)
        pltpu.async_copy(dst_hbm.at[pl.ds(base, CH)], dstv, st)
        pltpu.async_copy(a_hbm.at[pl.ds(base, CH)], av, st)

    def drain_scatters(slot):
        sc = slots[slot][8]
        for _ in range(3 * (CH // 128)):
            pltpu.make_async_copy(srcB.at[pl.ds(0, 128)], dmy, sc).wait()

    def run_chunk(k, ci, slot):
        srcv, dstv, av, posv, dlv, srcs, avs, st, sc = slots[slot]

        @pl.when(k > 0)
        def _():
            drain_scatters(slot)

        base = wid * EPT + ci * CH
        pltpu.make_async_copy(src_hbm.at[pl.ds(base, CH)], srcv, st).wait()
        pltpu.make_async_copy(dst_hbm.at[pl.ds(base, CH)], dstv, st).wait()
        pltpu.make_async_copy(a_hbm.at[pl.ds(base, CH)], av, st).wait()
        for g in range(CH // 16):
            d = dstv[pl.ds(g * 16, 16)]
            b = _bucket_of(d)
            cnt = jnp.zeros((16,), jnp.int32)
            rank = jnp.zeros((16,), jnp.int32)
            for j in range(16):
                eq = b == b[j]
                cnt = cnt + jnp.where(eq, 1, 0)
                rank = rank + jnp.where(eq & (iota > j), 1, 0)
            uniq = rank == 0
            basev = plsc.load_gather(offs, [b])
            plsc.store_scatter(offs, [b], basev + cnt, mask=uniq)
            posv[g // 8, pl.ds((g % 8) * 16, 16)] = basev + rank
            dlv[pl.ds(g * 16, 16)] = d - b * NPB
            srcs[pl.ds(g * 16, 16)] = srcv[pl.ds(g * 16, 16)]
            avs[pl.ds(g * 16, 16)] = av[pl.ds(g * 16, 16)]
        for j in range(CH // 128):
            sl = pl.ds(j * 128, 128)
            pltpu.async_copy(srcs.at[sl], srcB.at[posv.at[j]], sc)
            pltpu.async_copy(dlv.at[sl], dstB.at[posv.at[j]], sc)
            pltpu.async_copy(avs.at[sl], aB.at[posv.at[j]], sc)

        @pl.when(ci + 2 < NCH)
        def _():
            issue_stage(ci + 2, slot)

    issue_stage(jnp.int32(0), 0)
    issue_stage(jnp.int32(1), 1)

    def pair(k, c):
        run_chunk(k, k * 2, 0)
        run_chunk(k, k * 2 + 1, 1)
        return c

    lax.fori_loop(0, NCH // 2, pair, 0)
    drain_scatters(0)
    drain_scatters(1)


_ACC = (NPB + 1) * 32
_BPT = NB // 32


@functools.partial(
    pl.kernel,
    mesh=_mesh,
    compiler_params=_sc_params,
    out_type=[jax.ShapeDtypeStruct((NPAD * 32,), jnp.float32)
              for _ in range(4)],
    scratch_types=[
        pltpu.VMEM((32, NB), jnp.int32),
        pltpu.VMEM((NB,), jnp.int32),
        pltpu.VMEM((NB,), jnp.int32),
        pltpu.VMEM((CH,), jnp.int32),
        pltpu.VMEM((CH,), jnp.int32),
        pltpu.VMEM((CH,), jnp.float32),
        pltpu.VMEM((CH, 32), jnp.float32),
        pltpu.VMEM((CH,), jnp.int32),
        pltpu.VMEM((CH,), jnp.int32),
        pltpu.VMEM((CH,), jnp.float32),
        pltpu.VMEM((CH, 32), jnp.float32),
        pltpu.VMEM((32,), jnp.float32),
        pltpu.VMEM((_ACC,), jnp.float32),
        pltpu.VMEM((_ACC,), jnp.float32),
        pltpu.VMEM((_ACC,), jnp.float32),
        pltpu.VMEM((_ACC,), jnp.float32),
        pltpu.SemaphoreType.DMA,
        pltpu.SemaphoreType.DMA,
        pltpu.SemaphoreType.DMA,
        pltpu.SemaphoreType.DMA,
    ],
)
def _k_edge(srcB, dstB, aB, hist_hbm, u_hbm, q_hbm,
            sum_o, sq_o, mn_o, mx_o,
            histv, starts, sizes, srcv0, dstv0, av0, rows0,
            srcv1, dstv1, av1, rows1, uvv,
            accS, accQ, accN, accX, st0, st1, sg0, sg1):
    wid = _wid()
    pltpu.sync_copy(hist_hbm, histv)
    _bucket_offsets(histv, starts, sizes, wid, False)
    pltpu.sync_copy(u_hbm, uvv)
    u0 = uvv[pl.ds(0, 16)]
    u1 = uvv[pl.ds(16, 16)]
    iota = lax.iota(jnp.int32, 16)
    zf = jnp.zeros((16,), jnp.float32)
    bigv = jnp.full((16,), BIG, jnp.float32)
    slots = ((srcv0, dstv0, av0, rows0, st0, sg0),
             (srcv1, dstv1, av1, rows1, st1, sg1))

    for bi in range(_BPT):
        b = wid + bi * 32
        bvec = jnp.full((16,), b, jnp.int32)
        sb = plsc.load_gather(starts, [bvec])[0]
        tb = plsc.load_gather(sizes, [bvec])[0]
        nch = (tb + (CH - 1)) >> 9

        def init(i, c):
            accS[pl.ds(i * 16, 16)] = zf
            accQ[pl.ds(i * 16, 16)] = zf
            accN[pl.ds(i * 16, 16)] = bigv
            accX[pl.ds(i * 16, 16)] = -bigv
            return c

        lax.fori_loop(0, _ACC // 16, init, 0)

        def issue_stage(ci, slot):
            srcv, dstv, av, _, st, _ = slots[slot]

            @pl.when(ci < nch)
            def _():
                cbase = pl.multiple_of(sb + ci * CH, 16)
                pltpu.async_copy(srcB.at[pl.ds(cbase, CH)], srcv, st)
                pltpu.async_copy(dstB.at[pl.ds(cbase, CH)], dstv, st)
                pltpu.async_copy(aB.at[pl.ds(cbase, CH)], av, st)

        def run_chunk(ci, slot, nxt_ci, nxt_slot):
            srcv, dstv, av, rows, st, sg = slots[slot]

            @pl.when(ci < nch)
            def _():
                cbase = pl.multiple_of(sb + ci * CH, 16)
                pltpu.make_async_copy(srcB.at[pl.ds(cbase, CH)], srcv, st).wait()
                pltpu.make_async_copy(dstB.at[pl.ds(cbase, CH)], dstv, st).wait()
                pltpu.make_async_copy(aB.at[pl.ds(cbase, CH)], av, st).wait()

                def san(g, cc):
                    valid = (ci * CH + g * 16 + iota) < tb
                    sv = srcv[pl.ds(g * 16, 16)]
                    srcv[pl.ds(g * 16, 16)] = jnp.where(valid, sv, 0)
                    dv = dstv[pl.ds(g * 16, 16)]
                    dstv[pl.ds(g * 16, 16)] = jnp.where(valid, dv, NPB)
                    avv = av[pl.ds(g * 16, 16)]
                    av[pl.ds(g * 16, 16)] = jnp.where(valid, avv, 0.0)
                    return cc

                lax.fori_loop(0, CH // 16, san, 0)
                gathers = []
                for k in range(CH // 128):
                    gathers.append(pltpu.async_copy(
                        q_hbm.at[srcv.at[pl.ds(k * 128, 128)]],
                        rows.at[pl.ds(k * 128, 128)], sg))
                issue_stage(nxt_ci, nxt_slot)
                for gcp in gathers:
                    gcp.wait()

                def grp(g, cc):
                    dvec = dstv[pl.ds(g * 16, 16)]
                    avec = av[pl.ds(g * 16, 16)]
                    for j in range(16):
                        dloc = dvec[j]
                        aj = avec[j]
                        off = dloc * 32
                        eL = g * 16 + j
                        q0 = rows[eL, 0:16]
                        q1 = rows[eL, 16:32]
                        r0 = q0 + aj * u0
                        r1 = q1 + aj * u1
                        s0 = accS[pl.ds(off, 16)]
                        accS[pl.ds(off, 16)] = s0 + r0
                        s1 = accS[pl.ds(off + 16, 16)]
                        accS[pl.ds(off + 16, 16)] = s1 + r1
                        t0 = accQ[pl.ds(off, 16)]
                        accQ[pl.ds(off, 16)] = t0 + r0 * r0
                        t1 = accQ[pl.ds(off + 16, 16)]
                        accQ[pl.ds(off + 16, 16)] = t1 + r1 * r1
                        n0 = accN[pl.ds(off, 16)]
                        accN[pl.ds(off, 16)] = jnp.minimum(n0, r0)
                        n1 = accN[pl.ds(off + 16, 16)]
                        accN[pl.ds(off + 16, 16)] = jnp.minimum(n1, r1)
                        x0 = accX[pl.ds(off, 16)]
                        accX[pl.ds(off, 16)] = jnp.maximum(x0, r0)
                        x1 = accX[pl.ds(off + 16, 16)]
                        accX[pl.ds(off + 16, 16)] = jnp.maximum(x1, r1)
                    return cc

                lax.fori_loop(0, CH // 16, grp, 0)

        issue_stage(jnp.int32(0), 0)

        def pair(k, c):
            ci0 = k * 2
            run_chunk(ci0, 0, ci0 + 1, 1)
            run_chunk(ci0 + 1, 1, ci0 + 2, 0)
            return c

        lax.fori_loop(0, (nch + 1) >> 1, pair, 0)

        wout = NPB * 32
        obase = pl.multiple_of(b * wout, 16)
        pltpu.sync_copy(accS.at[pl.ds(0, wout)], sum_o.at[pl.ds(obase, wout)])
        pltpu.sync_copy(accQ.at[pl.ds(0, wout)], sq_o.at[pl.ds(obase, wout)])
        pltpu.sync_copy(accN.at[pl.ds(0, wout)], mn_o.at[pl.ds(obase, wout)])
        pltpu.sync_copy(accX.at[pl.ds(0, wout)], mx_o.at[pl.ds(obase, wout)])


def _dot(a, b):
    return jnp.dot(a, b, precision=lax.Precision.HIGHEST,
                   preferred_element_type=jnp.float32)


def _tcpre1_body(x_ref, wb_ref, q_ref):
    onehot = (lax.broadcasted_iota(jnp.int32, (1, 32), 1) == 25).astype(jnp.float32)
    q_ref[...] = _dot(x_ref[...], wb_ref[...]) + onehot


def _tcpre2_body(o_ref, st_ref, g_ref, be_ref, wb_ref, h_ref, q_ref):
    mu = st_ref[0:1, :]
    var = st_ref[1:2, :]
    h = jax.nn.relu((o_ref[...] - mu) * lax.rsqrt(var + 1e-5) * g_ref[...]
                    + be_ref[...])
    h_ref[...] = h
    onehot = (lax.broadcasted_iota(jnp.int32, (1, 32), 1) == 25).astype(jnp.float32)
    q_ref[...] = _dot(h, wb_ref[...]) + onehot


def _tcpost_body(h_ref, sum_ref, sq_ref, mn_ref, mx_ref, wa_ref, vb_ref,
                 wf1_ref, wf2_ref, wf3_ref, bias_ref, out_ref, st_ref, scr):
    i = pl.program_id(0)
    h = h_ref[...]
    s = _dot(h, wa_ref[...]) + vb_ref[...]
    cnt = sum_ref[:, 25:26]
    cnt_c = jnp.maximum(cnt, 1.0)
    has = cnt > 0.0
    sums = sum_ref[:, 0:25]
    sqs = sq_ref[:, 0:25]
    mean = jnp.where(has, s + sums / cnt_c, 0.0)
    mn = jnp.where(has, s + mn_ref[:, 0:25], 0.0)
    mx = jnp.where(has, s + mx_ref[:, 0:25], 0.0)
    var = sqs / cnt_c - (sums / cnt_c) ** 2
    std = jnp.sqrt(jax.nn.relu(var) + 1e-5)
    x_cat = jnp.concatenate([h, mean, mn, mx, std], axis=1)
    y_cat = jnp.concatenate([mean, mn, mx, std], axis=1)
    lg = jnp.log(cnt_c + 1.0)
    o = (_dot(x_cat, wf1_ref[...]) + (lg / AVG_LOG) * _dot(y_cat, wf2_ref[...])
         + (AVG_LOG / lg) * _dot(y_cat, wf3_ref[...]) + bias_ref[...])
    out_ref[...] = o

    @pl.when(i == 0)
    def _():
        scr[...] = jnp.zeros_like(scr)

    scr[0, 0:5] += jnp.sum(o, axis=0)
    scr[1, 0:5] += jnp.sum(o * o, axis=0)

    @pl.when(i == NBLK - 1)
    def _():
        mu = scr[0:1, 0:5] / N
        ex2 = scr[1:2, 0:5] / N
        st_ref[...] = jnp.concatenate([mu, ex2 - mu * mu], axis=0)


def _tcfinal_body(o_ref, st_ref, g_ref, be_ref, batch_ref,
                  w1_ref, b1_ref, w2_ref, b2_ref, w3_ref, b3_ref,
                  out_ref, scr):
    i = pl.program_id(0)
    mu = st_ref[0:1, :]
    var = st_ref[1:2, :]
    h = jax.nn.relu((o_ref[...] - mu) * lax.rsqrt(var + 1e-5) * g_ref[...]
                    + be_ref[...])
    seg = batch_ref[0, 0, :]
    onehot = (seg[:, None] == lax.broadcasted_iota(jnp.int32, (BLK, NG), 1)
              ).astype(jnp.float32)
    pooled = lax.dot_general(onehot, h, (((0,), (0,)), ((), ())),
                             precision=lax.Precision.HIGHEST,
                             preferred_element_type=jnp.float32)

    @pl.when(i == 0)
    def _():
        scr[...] = jnp.zeros_like(scr)

    scr[:, 0:5] += pooled

    @pl.when(i == NBLK - 1)
    def _():
        p = scr[:, 0:5]
        z1 = jax.nn.relu(_dot(p, w1_ref[...]) + b1_ref[...])
        z2 = jax.nn.relu(_dot(z1, w2_ref[...]) + b2_ref[...])
        out_ref[...] = _dot(z2, w3_ref[...]) + b3_ref[...]


def _row_spec(cols):
    return pl.BlockSpec((BLK, cols), lambda i: (i, 0))


def _full_spec(shape):
    nd = len(shape)
    return pl.BlockSpec(shape, lambda i: (0,) * nd)


def kernel(x, edge_index, edge_attr, batch, Wpre, bpre, Wedge, bedge, Wpost,
           bpost, Wlin, blin, bn_gamma, bn_beta, W1, b1, W2, b2, W3, b3):
    f32 = jnp.float32
    src = edge_index[0]
    dst = edge_index[1]
    a = edge_attr[:, 0]
    npad = EPAD - E
    srcp = jnp.concatenate([src, jnp.zeros((npad,), jnp.int32)])
    dstp = jnp.concatenate([dst, jnp.full((npad,), NPAD - 1, jnp.int32)])
    ap = jnp.concatenate([a, jnp.zeros((npad,), f32)])

    hist = _k_hist(dstp)
    srcB, dstB, aB = _k_scatter(srcp, dstp, ap, hist)

    eye_mask = jnp.asarray(np.kron(np.eye(T), np.ones((F, 1))), f32)

    def fold(ws):
        return ws.reshape(TF, 1) * eye_mask

    def layer_weights(l):
        Wp = Wpre[l]
        WA = Wp[:, 0:F, :].transpose(1, 0, 2).reshape(F, TF)
        WB = Wp[:, F:2 * F, :].transpose(1, 0, 2).reshape(F, TF)
        WC = Wp[:, 2 * F:3 * F, :].transpose(1, 0, 2).reshape(F, TF)
        u = Wedge[l][0] @ WC
        vb = bedge[l] @ WC + bpre[l].reshape(TF)
        upad = jnp.concatenate([u, jnp.zeros((7,), f32)])
        WBpad = jnp.concatenate([WB, jnp.zeros((F, 7), f32)], axis=1)
        Wp2 = Wpost[l][:, :, 0]
        wh = Wp2[:, 0:F]
        folds = [fold(Wp2[:, F + k * F:F + (k + 1) * F]) for k in range(12)]
        Wf1 = jnp.concatenate([wh.T] + folds[0:4], axis=0) @ Wlin[l]
        Wf2 = jnp.concatenate(folds[4:8], axis=0) @ Wlin[l]
        Wf3 = jnp.concatenate(folds[8:12], axis=0) @ Wlin[l]
        bias = (bpost[l][:, 0] @ Wlin[l] + blin[l]).reshape(1, F)
        return WA, WBpad, upad, vb.reshape(1, TF), Wf1, Wf2, Wf3, bias

    def run_edge_phase(qpad, upad):
        outs = _k_edge(srcB, dstB, aB, hist, upad, qpad)
        return [o.reshape(NPAD, 32)[:N] for o in outs]

    def tcpost(h, stats4, WA, vb, Wf1, Wf2, Wf3, bias):
        s_, q_, n_, x_ = stats4
        return pl.pallas_call(
            _tcpost_body,
            grid=(NBLK,),
            in_specs=[_row_spec(5), _row_spec(32), _row_spec(32),
                      _row_spec(32), _row_spec(32), _full_spec((F, TF)),
                      _full_spec((1, TF)), _full_spec((105, 5)),
                      _full_spec((100, 5)), _full_spec((100, 5)),
                      _full_spec((1, 5))],
            out_specs=[_row_spec(5), _full_spec((2, 5))],
            out_shape=[jax.ShapeDtypeStruct((N, 5), f32),
                       jax.ShapeDtypeStruct((2, 5), f32)],
            scratch_shapes=[pltpu.VMEM((8, 128), f32)],
        )(h, s_, q_, n_, x_, WA, vb, Wf1, Wf2, Wf3, bias)

    WA1, WBpad1, upad1, vb1, Wf11, Wf21, Wf31, bias1 = layer_weights(0)
    qpad1 = pl.pallas_call(
        _tcpre1_body,
        grid=(NBLK,),
        in_specs=[_row_spec(5), _full_spec((F, 32))],
        out_specs=_row_spec(32),
        out_shape=jax.ShapeDtypeStruct((N, 32), f32),
    )(x, WBpad1)
    st4_1 = run_edge_phase(qpad1, upad1)
    out1, bstats1 = tcpost(x, st4_1, WA1, vb1, Wf11, Wf21, Wf31, bias1)

    WA2, WBpad2, upad2, vb2, Wf12, Wf22, Wf32, bias2 = layer_weights(1)
    h2, qpad2 = pl.pallas_call(
        _tcpre2_body,
        grid=(NBLK,),
        in_specs=[_row_spec(5), _full_spec((2, 5)), _full_spec((1, 5)),
                  _full_spec((1, 5)), _full_spec((F, 32))],
        out_specs=[_row_spec(5), _row_spec(32)],
        out_shape=[jax.ShapeDtypeStruct((N, 5), f32),
                   jax.ShapeDtypeStruct((N, 32), f32)],
    )(out1, bstats1, bn_gamma[0].reshape(1, 5), bn_beta[0].reshape(1, 5),
      WBpad2)
    st4_2 = run_edge_phase(qpad2, upad2)
    out2, bstats2 = tcpost(h2, st4_2, WA2, vb2, Wf12, Wf22, Wf32, bias2)

    batch3d = batch.reshape(NBLK, 1, BLK)
    z = pl.pallas_call(
        _tcfinal_body,
        grid=(NBLK,),
        in_specs=[_row_spec(5), _full_spec((2, 5)), _full_spec((1, 5)),
                  _full_spec((1, 5)),
                  pl.BlockSpec((1, 1, BLK), lambda i: (i, 0, 0)),
                  _full_spec((5, 5)), _full_spec((1, 5)),
                  _full_spec((5, 10)), _full_spec((1, 10)),
                  _full_spec((10, 10)), _full_spec((1, 10))],
        out_specs=_full_spec((NG, 10)),
        out_shape=jax.ShapeDtypeStruct((NG, 10), f32),
        scratch_shapes=[pltpu.VMEM((NG, 128), f32)],
    )(out2, bstats2, bn_gamma[1].reshape(1, 5), bn_beta[1].reshape(1, 5),
      batch3d, W1, b1.reshape(1, 5), W2, b2.reshape(1, 10), W3,
      b3.reshape(1, 10))
    return z


# speedup vs baseline: 148.4861x; 1.1796x over previous
import functools
import math

import jax
import jax.numpy as jnp
import numpy as np
from jax import lax
from jax.experimental import pallas as pl
from jax.experimental.pallas import tpu as pltpu, tpu_sc as plsc

N = 50000
E = 800000
NG = 512
F = 5
T = 5
TF = T * F

_DEG = np.array([0, 0, 0, 0, 0, 0, 200, 400, 800, 1200, 1800, 2400, 3000,
                 3600, 4000, 4300, 4400, 4400, 4300, 4000, 3600, 3000, 2400,
                 1800, 1200, 800, 400, 200], dtype=np.float64)
AVG_LOG = float((np.log(np.arange(_DEG.shape[0]) + 1.0) * _DEG).sum() / _DEG.sum())

NB = 128
NPB = 392
NPAD = NB * NPB
EPT = 25600
EPAD = 32 * EPT
CAP = 821760
CH = 512
BLK = 2000
NBLK = 25
BIG = 3.0e38

_mesh = plsc.VectorSubcoreMesh(core_axis_name="c", subcore_axis_name="s")
_sc_params = pltpu.CompilerParams(
    needs_layout_passes=False, use_tc_tiling_on_sc=False)


def _wid():
    return lax.axis_index("s") * 2 + lax.axis_index("c")


def _bucket_of(d):
    return ((d >> 3) * 2675) >> 17


def _bucket_offsets(histv, offs, sizes, wid, with_pre):
    carry = jnp.int32(0)
    zero16 = jnp.zeros((16,), jnp.int32)
    for g in range(NB // 16):
        tot = zero16
        pre = zero16
        for t in range(32):
            hrow = histv[t, g * 16:(g + 1) * 16]
            tot = tot + hrow
            if with_pre:
                pre = pre + jnp.where(t < wid, hrow, zero16)
        padded = (tot + 15) & (-16)
        incl = plsc.cumsum(padded)
        excl = incl - padded + carry
        if with_pre:
            offs[pl.ds(g * 16, 16)] = excl + pre
        else:
            offs[pl.ds(g * 16, 16)] = excl
        if sizes is not None:
            sizes[pl.ds(g * 16, 16)] = tot
        carry = carry + incl[15]


@functools.partial(
    pl.kernel,
    mesh=_mesh,
    compiler_params=_sc_params,
    out_type=jax.ShapeDtypeStruct((32, NB), jnp.int32),
    scratch_types=[
        pltpu.VMEM((512,), jnp.int32),
        pltpu.VMEM((NB,), jnp.int32),
    ],
)
def _k_hist(dst_hbm, out_hbm, dstv, hist):
    wid = _wid()

    def z(i, c):
        hist[pl.ds(i * 16, 16)] = jnp.zeros((16,), jnp.int32)
        return c

    lax.fori_loop(0, NB // 16, z, 0)
    iota = lax.iota(jnp.int32, 16)

    def chunk(ci, c):
        pltpu.sync_copy(dst_hbm.at[pl.ds(wid * EPT + ci * 512, 512)], dstv)

        def grp(g, cc):
            d = dstv[pl.ds(g * 16, 16)]
            b = _bucket_of(d)
            cnt = jnp.zeros((16,), jnp.int32)
            rank = jnp.zeros((16,), jnp.int32)
            for j in range(16):
                eq = b == b[j]
                cnt = cnt + jnp.where(eq, 1, 0)
                rank = rank + jnp.where(eq & (iota > j), 1, 0)
            uniq = rank == 0
            old = plsc.load_gather(hist, [b], mask=uniq)
            plsc.store_scatter(hist, [b], old + cnt, mask=uniq)
            return cc

        lax.fori_loop(0, 32, grp, 0)
        return c

    lax.fori_loop(0, EPT // 512, chunk, 0)
    pltpu.sync_copy(hist, out_hbm.at[wid])


@functools.partial(
    pl.kernel,
    mesh=_mesh,
    compiler_params=_sc_params,
    out_type=[
        jax.ShapeDtypeStruct((CAP,), jnp.int32),
        jax.ShapeDtypeStruct((CAP,), jnp.float32),
    ],
    scratch_types=[
        pltpu.VMEM((32, NB), jnp.int32),
        pltpu.VMEM((NB,), jnp.int32),
        pltpu.VMEM((CH,), jnp.int32),
        pltpu.VMEM((CH,), jnp.int32),
        pltpu.VMEM((CH,), jnp.float32),
        pltpu.VMEM((CH,), jnp.int32),
        pltpu.VMEM((CH,), jnp.int32),
        pltpu.VMEM((CH,), jnp.float32),
        pltpu.VMEM((CH // 128, 128), jnp.int32),
        pltpu.VMEM((CH // 128, 128), jnp.int32),
        pltpu.VMEM((CH,), jnp.int32),
        pltpu.VMEM((CH,), jnp.int32),
        pltpu.VMEM((CH,), jnp.float32),
        pltpu.VMEM((CH,), jnp.float32),
        pltpu.VMEM((128,), jnp.int32),
        pltpu.SemaphoreType.DMA,
        pltpu.SemaphoreType.DMA,
        pltpu.SemaphoreType.DMA,
        pltpu.SemaphoreType.DMA,
    ],
)
def _k_scatter(src_hbm, dst_hbm, a_hbm, hist_hbm, pkB, aB,
               histv, offs, srcv0, dstv0, av0, srcv1, dstv1, av1,
               posv0, posv1, dlv0, dlv1, avs0, avs1,
               dmy, st0, st1, sc0, sc1):
    wid = _wid()
    pltpu.sync_copy(hist_hbm, histv)
    _bucket_offsets(histv, offs, None, wid, True)
    iota = lax.iota(jnp.int32, 16)
    NCH = EPT // CH
    slots = ((srcv0, dstv0, av0, posv0, dlv0, avs0, st0, sc0),
             (srcv1, dstv1, av1, posv1, dlv1, avs1, st1, sc1))

    def issue_stage(ci, slot):
        srcv, dstv, av = slots[slot][0:3]
        st = slots[slot][6]
        base = wid * EPT + ci * CH
        pltpu.async_copy(src_hbm.at[pl.ds(base, CH)], srcv, st)
        pltpu.async_copy(dst_hbm.at[pl.ds(base, CH)], dstv, st)
        pltpu.async_copy(a_hbm.at[pl.ds(base, CH)], av, st)

    def drain_scatters(slot):
        sc = slots[slot][7]
        for _ in range(2 * (CH // 128)):
            pltpu.make_async_copy(pkB.at[pl.ds(0, 128)], dmy, sc).wait()

    def run_chunk(k, ci, slot):
        srcv, dstv, av, posv, dlv, avs, st, sc = slots[slot]

        @pl.when(k > 0)
        def _():
            drain_scatters(slot)

        base = wid * EPT + ci * CH
        pltpu.make_async_copy(src_hbm.at[pl.ds(base, CH)], srcv, st).wait()
        pltpu.make_async_copy(dst_hbm.at[pl.ds(base, CH)], dstv, st).wait()
        pltpu.make_async_copy(a_hbm.at[pl.ds(base, CH)], av, st).wait()
        for g in range(CH // 16):
            d = dstv[pl.ds(g * 16, 16)]
            b = _bucket_of(d)
            cnt = jnp.zeros((16,), jnp.int32)
            rank = jnp.zeros((16,), jnp.int32)
            for j in range(16):
                eq = b == b[j]
                cnt = cnt + jnp.where(eq, 1, 0)
                rank = rank + jnp.where(eq & (iota > j), 1, 0)
            uniq = rank == 0
            basev = plsc.load_gather(offs, [b])
            plsc.store_scatter(offs, [b], basev + cnt, mask=uniq)
            posv[g // 8, pl.ds((g % 8) * 16, 16)] = basev + rank
            dlv[pl.ds(g * 16, 16)] = (srcv[pl.ds(g * 16, 16)]
                                      + ((d - b * NPB) << 16))
            avs[pl.ds(g * 16, 16)] = av[pl.ds(g * 16, 16)]
        for j in range(CH // 128):
            sl = pl.ds(j * 128, 128)
            pltpu.async_copy(dlv.at[sl], pkB.at[posv.at[j]], sc)
            pltpu.async_copy(avs.at[sl], aB.at[posv.at[j]], sc)

        @pl.when(ci + 2 < NCH)
        def _():
            issue_stage(ci + 2, slot)

    issue_stage(jnp.int32(0), 0)
    issue_stage(jnp.int32(1), 1)

    def pair(k, c):
        run_chunk(k, k * 2, 0)
        run_chunk(k, k * 2 + 1, 1)
        return c

    lax.fori_loop(0, NCH // 2, pair, 0)
    drain_scatters(0)
    drain_scatters(1)


_ACC = (NPB + 1) * 32
_BPT = NB // 32


@functools.partial(
    pl.kernel,
    mesh=_mesh,
    compiler_params=_sc_params,
    out_type=[jax.ShapeDtypeStruct((NPAD * 32,), jnp.float32)
              for _ in range(4)],
    scratch_types=[
        pltpu.VMEM((32, NB), jnp.int32),
        pltpu.VMEM((NB,), jnp.int32),
        pltpu.VMEM((NB,), jnp.int32),
        pltpu.VMEM((CH,), jnp.int32),
        pltpu.VMEM((CH,), jnp.int32),
        pltpu.VMEM((CH,), jnp.float32),
        pltpu.VMEM((CH, 32), jnp.float32),
        pltpu.VMEM((CH,), jnp.int32),
        pltpu.VMEM((CH,), jnp.int32),
        pltpu.VMEM((CH,), jnp.float32),
        pltpu.VMEM((CH, 32), jnp.float32),
        pltpu.VMEM((32,), jnp.float32),
        pltpu.VMEM((_ACC,), jnp.float32),
        pltpu.VMEM((_ACC,), jnp.float32),
        pltpu.VMEM((_ACC,), jnp.float32),
        pltpu.VMEM((_ACC,), jnp.float32),
        pltpu.SemaphoreType.DMA,
        pltpu.SemaphoreType.DMA,
        pltpu.SemaphoreType.DMA,
        pltpu.SemaphoreType.DMA,
    ],
)
def _k_edge(pkB, aB, hist_hbm, u_hbm, q_hbm,
            sum_o, sq_o, mn_o, mx_o,
            histv, starts, sizes, srcv0, dstv0, av0, rows0,
            srcv1, dstv1, av1, rows1, uvv,
            accS, accQ, accN, accX, st0, st1, sg0, sg1):
    wid = _wid()
    pltpu.sync_copy(hist_hbm, histv)
    _bucket_offsets(histv, starts, sizes, wid, False)
    pltpu.sync_copy(u_hbm, uvv)
    u0 = uvv[pl.ds(0, 16)]
    u1 = uvv[pl.ds(16, 16)]
    iota = lax.iota(jnp.int32, 16)
    zf = jnp.zeros((16,), jnp.float32)
    bigv = jnp.full((16,), BIG, jnp.float32)
    slots = ((srcv0, dstv0, av0, rows0, st0, sg0),
             (srcv1, dstv1, av1, rows1, st1, sg1))

    for bi in range(_BPT):
        b = wid + bi * 32
        bvec = jnp.full((16,), b, jnp.int32)
        sb = plsc.load_gather(starts, [bvec])[0]
        tb = plsc.load_gather(sizes, [bvec])[0]
        nch = (tb + (CH - 1)) >> 9

        def init(i, c):
            accS[pl.ds(i * 16, 16)] = zf
            accQ[pl.ds(i * 16, 16)] = zf
            accN[pl.ds(i * 16, 16)] = bigv
            accX[pl.ds(i * 16, 16)] = -bigv
            return c

        lax.fori_loop(0, _ACC // 16, init, 0)

        def issue_stage(ci, slot):
            srcv, dstv, av, _, st, _ = slots[slot]

            @pl.when(ci < nch)
            def _():
                cbase = pl.multiple_of(sb + ci * CH, 16)
                pltpu.async_copy(pkB.at[pl.ds(cbase, CH)], dstv, st)
                pltpu.async_copy(aB.at[pl.ds(cbase, CH)], av, st)

        def run_chunk(ci, slot, nxt_ci, nxt_slot):
            srcv, dstv, av, rows, st, sg = slots[slot]

            @pl.when(ci < nch)
            def _():
                cbase = pl.multiple_of(sb + ci * CH, 16)
                pltpu.make_async_copy(pkB.at[pl.ds(cbase, CH)], dstv, st).wait()
                pltpu.make_async_copy(aB.at[pl.ds(cbase, CH)], av, st).wait()

                def san(g, cc):
                    valid = (ci * CH + g * 16 + iota) < tb
                    pk = dstv[pl.ds(g * 16, 16)]
                    srcv[pl.ds(g * 16, 16)] = jnp.where(valid, pk & 0xFFFF, 0)
                    dstv[pl.ds(g * 16, 16)] = jnp.where(valid, pk >> 16, NPB)
                    avv = av[pl.ds(g * 16, 16)]
                    av[pl.ds(g * 16, 16)] = jnp.where(valid, avv, 0.0)
                    return cc

                lax.fori_loop(0, CH // 16, san, 0)
                gathers = []
                for k in range(CH // 128):
                    gathers.append(pltpu.async_copy(
                        q_hbm.at[srcv.at[pl.ds(k * 128, 128)]],
                        rows.at[pl.ds(k * 128, 128)], sg))
                issue_stage(nxt_ci, nxt_slot)
                for gcp in gathers:
                    gcp.wait()

                def grp(g, cc):
                    dvec = dstv[pl.ds(g * 16, 16)]
                    avec = av[pl.ds(g * 16, 16)]
                    for j in range(16):
                        dloc = dvec[j]
                        aj = avec[j]
                        off = dloc * 32
                        eL = g * 16 + j
                        q0 = rows[eL, 0:16]
                        q1 = rows[eL, 16:32]
                        r0 = q0 + aj * u0
                        r1 = q1 + aj * u1
                        s0 = accS[pl.ds(off, 16)]
                        accS[pl.ds(off, 16)] = s0 + r0
                        s1 = accS[pl.ds(off + 16, 16)]
                        accS[pl.ds(off + 16, 16)] = s1 + r1
                        t0 = accQ[pl.ds(off, 16)]
                        accQ[pl.ds(off, 16)] = t0 + r0 * r0
                        t1 = accQ[pl.ds(off + 16, 16)]
                        accQ[pl.ds(off + 16, 16)] = t1 + r1 * r1
                        n0 = accN[pl.ds(off, 16)]
                        accN[pl.ds(off, 16)] = jnp.minimum(n0, r0)
                        n1 = accN[pl.ds(off + 16, 16)]
                        accN[pl.ds(off + 16, 16)] = jnp.minimum(n1, r1)
                        x0 = accX[pl.ds(off, 16)]
                        accX[pl.ds(off, 16)] = jnp.maximum(x0, r0)
                        x1 = accX[pl.ds(off + 16, 16)]
                        accX[pl.ds(off + 16, 16)] = jnp.maximum(x1, r1)
                    return cc

                lax.fori_loop(0, CH // 16, grp, 0)

        issue_stage(jnp.int32(0), 0)

        def pair(k, c):
            ci0 = k * 2
            run_chunk(ci0, 0, ci0 + 1, 1)
            run_chunk(ci0 + 1, 1, ci0 + 2, 0)
            return c

        lax.fori_loop(0, (nch + 1) >> 1, pair, 0)

        wout = NPB * 32
        obase = pl.multiple_of(b * wout, 16)
        pltpu.sync_copy(accS.at[pl.ds(0, wout)], sum_o.at[pl.ds(obase, wout)])
        pltpu.sync_copy(accQ.at[pl.ds(0, wout)], sq_o.at[pl.ds(obase, wout)])
        pltpu.sync_copy(accN.at[pl.ds(0, wout)], mn_o.at[pl.ds(obase, wout)])
        pltpu.sync_copy(accX.at[pl.ds(0, wout)], mx_o.at[pl.ds(obase, wout)])


def _dot(a, b):
    return jnp.dot(a, b, precision=lax.Precision.HIGHEST,
                   preferred_element_type=jnp.float32)


def _tcpre1_body(x_ref, wb_ref, q_ref):
    onehot = (lax.broadcasted_iota(jnp.int32, (1, 32), 1) == 25).astype(jnp.float32)
    q_ref[...] = _dot(x_ref[...], wb_ref[...]) + onehot


def _tcpre2_body(o_ref, st_ref, g_ref, be_ref, wb_ref, h_ref, q_ref):
    mu = st_ref[0:1, :]
    var = st_ref[1:2, :]
    h = jax.nn.relu((o_ref[...] - mu) * lax.rsqrt(var + 1e-5) * g_ref[...]
                    + be_ref[...])
    h_ref[...] = h
    onehot = (lax.broadcasted_iota(jnp.int32, (1, 32), 1) == 25).astype(jnp.float32)
    q_ref[...] = _dot(h, wb_ref[...]) + onehot


def _tcpost_body(h_ref, sum_ref, sq_ref, mn_ref, mx_ref, wa_ref, vb_ref,
                 wf1_ref, wf2_ref, wf3_ref, bias_ref, out_ref, st_ref, scr):
    i = pl.program_id(0)
    h = h_ref[...]
    s = _dot(h, wa_ref[...]) + vb_ref[...]
    cnt = sum_ref[:, 25:26]
    cnt_c = jnp.maximum(cnt, 1.0)
    has = cnt > 0.0
    sums = sum_ref[:, 0:25]
    sqs = sq_ref[:, 0:25]
    mean = jnp.where(has, s + sums / cnt_c, 0.0)
    mn = jnp.where(has, s + mn_ref[:, 0:25], 0.0)
    mx = jnp.where(has, s + mx_ref[:, 0:25], 0.0)
    var = sqs / cnt_c - (sums / cnt_c) ** 2
    std = jnp.sqrt(jax.nn.relu(var) + 1e-5)
    x_cat = jnp.concatenate([h, mean, mn, mx, std], axis=1)
    y_cat = jnp.concatenate([mean, mn, mx, std], axis=1)
    lg = jnp.log(cnt_c + 1.0)
    o = (_dot(x_cat, wf1_ref[...]) + (lg / AVG_LOG) * _dot(y_cat, wf2_ref[...])
         + (AVG_LOG / lg) * _dot(y_cat, wf3_ref[...]) + bias_ref[...])
    out_ref[...] = o

    @pl.when(i == 0)
    def _():
        scr[...] = jnp.zeros_like(scr)

    scr[0, 0:5] += jnp.sum(o, axis=0)
    scr[1, 0:5] += jnp.sum(o * o, axis=0)

    @pl.when(i == NBLK - 1)
    def _():
        mu = scr[0:1, 0:5] / N
        ex2 = scr[1:2, 0:5] / N
        st_ref[...] = jnp.concatenate([mu, ex2 - mu * mu], axis=0)


def _tcfinal_body(o_ref, st_ref, g_ref, be_ref, batch_ref,
                  w1_ref, b1_ref, w2_ref, b2_ref, w3_ref, b3_ref,
                  out_ref, scr):
    i = pl.program_id(0)
    mu = st_ref[0:1, :]
    var = st_ref[1:2, :]
    h = jax.nn.relu((o_ref[...] - mu) * lax.rsqrt(var + 1e-5) * g_ref[...]
                    + be_ref[...])
    seg = batch_ref[0, 0, :]
    onehot = (seg[:, None] == lax.broadcasted_iota(jnp.int32, (BLK, NG), 1)
              ).astype(jnp.float32)
    pooled = lax.dot_general(onehot, h, (((0,), (0,)), ((), ())),
                             precision=lax.Precision.HIGHEST,
                             preferred_element_type=jnp.float32)

    @pl.when(i == 0)
    def _():
        scr[...] = jnp.zeros_like(scr)

    scr[:, 0:5] += pooled

    @pl.when(i == NBLK - 1)
    def _():
        p = scr[:, 0:5]
        z1 = jax.nn.relu(_dot(p, w1_ref[...]) + b1_ref[...])
        z2 = jax.nn.relu(_dot(z1, w2_ref[...]) + b2_ref[...])
        out_ref[...] = _dot(z2, w3_ref[...]) + b3_ref[...]


def _row_spec(cols):
    return pl.BlockSpec((BLK, cols), lambda i: (i, 0))


def _full_spec(shape):
    nd = len(shape)
    return pl.BlockSpec(shape, lambda i: (0,) * nd)


def kernel(x, edge_index, edge_attr, batch, Wpre, bpre, Wedge, bedge, Wpost,
           bpost, Wlin, blin, bn_gamma, bn_beta, W1, b1, W2, b2, W3, b3):
    f32 = jnp.float32
    src = edge_index[0]
    dst = edge_index[1]
    a = edge_attr[:, 0]
    npad = EPAD - E
    srcp = jnp.concatenate([src, jnp.zeros((npad,), jnp.int32)])
    dstp = jnp.concatenate([dst, jnp.full((npad,), NPAD - 1, jnp.int32)])
    ap = jnp.concatenate([a, jnp.zeros((npad,), f32)])

    hist = _k_hist(dstp)
    pkB, aB = _k_scatter(srcp, dstp, ap, hist)

    eye_mask = jnp.asarray(np.kron(np.eye(T), np.ones((F, 1))), f32)

    def fold(ws):
        return ws.reshape(TF, 1) * eye_mask

    def layer_weights(l):
        Wp = Wpre[l]
        WA = Wp[:, 0:F, :].transpose(1, 0, 2).reshape(F, TF)
        WB = Wp[:, F:2 * F, :].transpose(1, 0, 2).reshape(F, TF)
        WC = Wp[:, 2 * F:3 * F, :].transpose(1, 0, 2).reshape(F, TF)
        u = Wedge[l][0] @ WC
        vb = bedge[l] @ WC + bpre[l].reshape(TF)
        upad = jnp.concatenate([u, jnp.zeros((7,), f32)])
        WBpad = jnp.concatenate([WB, jnp.zeros((F, 7), f32)], axis=1)
        Wp2 = Wpost[l][:, :, 0]
        wh = Wp2[:, 0:F]
        folds = [fold(Wp2[:, F + k * F:F + (k + 1) * F]) for k in range(12)]
        Wf1 = jnp.concatenate([wh.T] + folds[0:4], axis=0) @ Wlin[l]
        Wf2 = jnp.concatenate(folds[4:8], axis=0) @ Wlin[l]
        Wf3 = jnp.concatenate(folds[8:12], axis=0) @ Wlin[l]
        bias = (bpost[l][:, 0] @ Wlin[l] + blin[l]).reshape(1, F)
        return WA, WBpad, upad, vb.reshape(1, TF), Wf1, Wf2, Wf3, bias

    def run_edge_phase(qpad, upad):
        outs = _k_edge(pkB, aB, hist, upad, qpad)
        return [o.reshape(NPAD, 32)[:N] for o in outs]

    def tcpost(h, stats4, WA, vb, Wf1, Wf2, Wf3, bias):
        s_, q_, n_, x_ = stats4
        return pl.pallas_call(
            _tcpost_body,
            grid=(NBLK,),
            in_specs=[_row_spec(5), _row_spec(32), _row_spec(32),
                      _row_spec(32), _row_spec(32), _full_spec((F, TF)),
                      _full_spec((1, TF)), _full_spec((105, 5)),
                      _full_spec((100, 5)), _full_spec((100, 5)),
                      _full_spec((1, 5))],
            out_specs=[_row_spec(5), _full_spec((2, 5))],
            out_shape=[jax.ShapeDtypeStruct((N, 5), f32),
                       jax.ShapeDtypeStruct((2, 5), f32)],
            scratch_shapes=[pltpu.VMEM((8, 128), f32)],
        )(h, s_, q_, n_, x_, WA, vb, Wf1, Wf2, Wf3, bias)

    WA1, WBpad1, upad1, vb1, Wf11, Wf21, Wf31, bias1 = layer_weights(0)
    qpad1 = pl.pallas_call(
        _tcpre1_body,
        grid=(NBLK,),
        in_specs=[_row_spec(5), _full_spec((F, 32))],
        out_specs=_row_spec(32),
        out_shape=jax.ShapeDtypeStruct((N, 32), f32),
    )(x, WBpad1)
    st4_1 = run_edge_phase(qpad1, upad1)
    out1, bstats1 = tcpost(x, st4_1, WA1, vb1, Wf11, Wf21, Wf31, bias1)

    WA2, WBpad2, upad2, vb2, Wf12, Wf22, Wf32, bias2 = layer_weights(1)
    h2, qpad2 = pl.pallas_call(
        _tcpre2_body,
        grid=(NBLK,),
        in_specs=[_row_spec(5), _full_spec((2, 5)), _full_spec((1, 5)),
                  _full_spec((1, 5)), _full_spec((F, 32))],
        out_specs=[_row_spec(5), _row_spec(32)],
        out_shape=[jax.ShapeDtypeStruct((N, 5), f32),
                   jax.ShapeDtypeStruct((N, 32), f32)],
    )(out1, bstats1, bn_gamma[0].reshape(1, 5), bn_beta[0].reshape(1, 5),
      WBpad2)
    st4_2 = run_edge_phase(qpad2, upad2)
    out2, bstats2 = tcpost(h2, st4_2, WA2, vb2, Wf12, Wf22, Wf32, bias2)

    batch3d = batch.reshape(NBLK, 1, BLK)
    z = pl.pallas_call(
        _tcfinal_body,
        grid=(NBLK,),
        in_specs=[_row_spec(5), _full_spec((2, 5)), _full_spec((1, 5)),
                  _full_spec((1, 5)),
                  pl.BlockSpec((1, 1, BLK), lambda i: (i, 0, 0)),
                  _full_spec((5, 5)), _full_spec((1, 5)),
                  _full_spec((5, 10)), _full_spec((1, 10)),
                  _full_spec((10, 10)), _full_spec((1, 10))],
        out_specs=_full_spec((NG, 10)),
        out_shape=jax.ShapeDtypeStruct((NG, 10), f32),
        scratch_shapes=[pltpu.VMEM((NG, 128), f32)],
    )(out2, bstats2, bn_gamma[1].reshape(1, 5), bn_beta[1].reshape(1, 5),
      batch3d, W1, b1.reshape(1, 5), W2, b2.reshape(1, 10), W3,
      b3.reshape(1, 10))
    return z
